# Initial kernel scaffold; baseline (speedup 1.0000x reference)
import jax
import jax.numpy as jnp
from jax.experimental import pallas as pl


def kernel(input1, input2):
    raise NotImplementedError("write your pallas kernel here")


# speedup vs baseline: 1.5520x; 1.5520x over previous
import functools

import jax
import jax.numpy as jnp
from jax.experimental import pallas as pl
from jax.experimental.pallas import tpu as pltpu

_B, _N, _M, _D = 4, 4096, 4096, 3
_TN = 256
_NB = _N // _TN


def _chamfer_block_kernel(x_ref, yt_ref, out_ref, m2_ref):
    b = pl.program_id(0)
    i = pl.program_id(1)

    x = x_ref[0]
    yt = yt_ref[0]

    d = None
    for k in range(_D):
        xk = x[:, k][:, None]
        ykt = yt[k, :][None, :]
        diff = xk - ykt
        sq = diff * diff
        d = sq if d is None else d + sq

    s1 = jnp.sum(jnp.min(d, axis=1), keepdims=True)[None, :]

    m2 = jnp.min(d, axis=0, keepdims=True)

    @pl.when(jnp.logical_and(b == 0, i == 0))
    def _init_out():
        out_ref[...] = jnp.zeros((1, 1), jnp.float32)

    @pl.when(i == 0)
    def _init_m2():
        m2_ref[...] = m2

    @pl.when(i > 0)
    def _acc_m2():
        m2_ref[...] = jnp.minimum(m2_ref[...], m2)

    out_ref[...] += s1 * (1.0 / (_B * _N))

    @pl.when(i == _NB - 1)
    def _flush_m2():
        out_ref[...] += jnp.sum(m2_ref[...], keepdims=True) * (1.0 / (_B * _M))


@jax.jit
def kernel(input1, input2):
    yt = jnp.transpose(input2, (0, 2, 1))
    out = pl.pallas_call(
        _chamfer_block_kernel,
        grid=(_B, _NB),
        in_specs=[
            pl.BlockSpec((1, _TN, _D), lambda b, i: (b, i, 0)),
            pl.BlockSpec((1, _D, _M), lambda b, i: (b, 0, 0)),
        ],
        out_specs=pl.BlockSpec((1, 1), lambda b, i: (0, 0)),
        out_shape=jax.ShapeDtypeStruct((1, 1), jnp.float32),
        scratch_shapes=[pltpu.VMEM((1, _M), jnp.float32)],
    )(input1, yt)
    return out[0, 0]


# speedup vs baseline: 1.7802x; 1.1470x over previous
import jax
import jax.numpy as jnp
from jax.experimental import pallas as pl
from jax.experimental.pallas import tpu as pltpu

_B, _N, _M, _D = 4, 4096, 4096, 3
_TN = 2048
_NB = _N // _TN


def _chamfer_block_kernel(x_ref, yt_ref, out_ref, m2_ref):
    b = pl.program_id(0)
    i = pl.program_id(1)

    x = x_ref[0]
    yt = yt_ref[0]

    d = None
    for k in range(_D):
        xk = x[:, k][:, None]
        ykt = yt[k, :][None, :]
        diff = xk - ykt
        sq = diff * diff
        d = sq if d is None else d + sq

    s1 = jnp.sum(jnp.min(d, axis=1), keepdims=True)[None, :]

    m2 = jnp.min(d, axis=0, keepdims=True)

    @pl.when(jnp.logical_and(b == 0, i == 0))
    def _init_out():
        out_ref[...] = jnp.zeros((1, 1), jnp.float32)

    @pl.when(i == 0)
    def _init_m2():
        m2_ref[...] = m2

    @pl.when(i > 0)
    def _acc_m2():
        m2_ref[...] = jnp.minimum(m2_ref[...], m2)

    out_ref[...] += s1 * (1.0 / (_B * _N))

    @pl.when(i == _NB - 1)
    def _flush_m2():
        out_ref[...] += jnp.sum(m2_ref[...], keepdims=True) * (1.0 / (_B * _M))


@jax.jit
def kernel(input1, input2):
    yt = jnp.transpose(input2, (0, 2, 1))
    out = pl.pallas_call(
        _chamfer_block_kernel,
        grid=(_B, _NB),
        in_specs=[
            pl.BlockSpec((1, _TN, _D), lambda b, i: (b, i, 0)),
            pl.BlockSpec((1, _D, _M), lambda b, i: (b, 0, 0)),
        ],
        out_specs=pl.BlockSpec((1, 1), lambda b, i: (0, 0)),
        out_shape=jax.ShapeDtypeStruct((1, 1), jnp.float32),
        scratch_shapes=[pltpu.VMEM((1, _M), jnp.float32)],
    )(input1, yt)
    return out[0, 0]


# speedup vs baseline: 1.7851x; 1.0027x over previous
import jax
import jax.numpy as jnp
from jax.experimental import pallas as pl
from jax.experimental.pallas import tpu as pltpu

_B, _N, _M, _D = 4, 4096, 4096, 3
_TN = 2048
_NB = _N // _TN


def _chamfer_block_kernel(u_ref, xn_ref, yt_ref, yn_ref, out_ref, m2_ref):
    b = pl.program_id(0)
    i = pl.program_id(1)

    u = u_ref[0]
    xn = xn_ref[0]
    yt = yt_ref[0]
    yn = yn_ref[0]

    g = None
    for k in range(_D):
        uk = u[:, k][:, None]
        ykt = yt[k, :][None, :]
        p = uk * ykt
        g = p if g is None else g + p

    t1 = g + yn
    r1 = jnp.min(t1, axis=1)[:, None] + xn
    s1 = jnp.sum(r1, keepdims=True)[:1, :1]

    t2 = g + xn
    m2 = jnp.min(t2, axis=0, keepdims=True)

    @pl.when(jnp.logical_and(b == 0, i == 0))
    def _init_out():
        out_ref[...] = jnp.zeros((1, 1), jnp.float32)

    @pl.when(i == 0)
    def _init_m2():
        m2_ref[...] = m2

    @pl.when(i > 0)
    def _acc_m2():
        m2_ref[...] = jnp.minimum(m2_ref[...], m2)

    out_ref[...] += s1 * (1.0 / (_B * _N))

    @pl.when(i == _NB - 1)
    def _flush_m2():
        out_ref[...] += jnp.sum(m2_ref[...] + yn, keepdims=True) * (
            1.0 / (_B * _M)
        )


@jax.jit
def kernel(input1, input2):
    u = -2.0 * input1
    xn = jnp.sum(input1 * input1, axis=2, keepdims=True)
    yt = jnp.transpose(input2, (0, 2, 1))
    yn = jnp.sum(input2 * input2, axis=2)[:, None, :]
    out = pl.pallas_call(
        _chamfer_block_kernel,
        grid=(_B, _NB),
        in_specs=[
            pl.BlockSpec((1, _TN, _D), lambda b, i: (b, i, 0)),
            pl.BlockSpec((1, _TN, 1), lambda b, i: (b, i, 0)),
            pl.BlockSpec((1, _D, _M), lambda b, i: (b, 0, 0)),
            pl.BlockSpec((1, 1, _M), lambda b, i: (b, 0, 0)),
        ],
        out_specs=pl.BlockSpec((1, 1), lambda b, i: (0, 0)),
        out_shape=jax.ShapeDtypeStruct((1, 1), jnp.float32),
        scratch_shapes=[pltpu.VMEM((1, _M), jnp.float32)],
    )(u, xn, yt, yn)
    return out[0, 0]
